# Initial kernel scaffold; baseline (speedup 1.0000x reference)
import jax
import jax.numpy as jnp
from jax.experimental import pallas as pl


def kernel(x_nchw, slab):
    raise NotImplementedError("write your pallas kernel here")


# speedup vs baseline: 2.3178x; 2.3178x over previous
import numpy as np
import jax
import jax.numpy as jnp
from jax import lax
from jax.experimental import pallas as pl
from jax.experimental.pallas import tpu as pltpu

NB = 8
_B = 1024
_GROWTH = 32
_INTER = 128

_SPECS = ((32, 32, 3, 65), (16, 16, 65, 96), (8, 8, 96, 112))
_HIDDEN = 256


def _rup(n, k):
    return -(-n // k) * k


def _layouts():
    row = [0]

    def add(r):
        off = row[0]
        row[0] += _rup(r, 8)
        return off

    w1_row = [0]
    vec_row = [0]

    def vadd(r):
        off = vec_row[0]
        vec_row[0] += r
        return off

    stages = []
    for si, (H, W, C_in, C_out) in enumerate(_SPECS):
        G0 = _rup(C_in, 8)
        CP = G0 + 4 * _GROWTH
        Cop = _rup(C_out, 8)
        blocks = []
        for j in range(4):
            Kj = G0 + j * _GROWTH
            blk = dict(
                Kj=Kj,
                r_s1=add(CP), r_t1=add(CP), r_w1=add(_INTER),
                r_s2=add(_INTER), r_t2=add(_INTER), r_w2=add(9 * _GROWTH),
                w1t=w1_row[0], w2t=_INTER * (si * 4 + j),
                v_s1=vadd(Kj), v_t1=vadd(Kj), v_t2=vadd(_INTER),
            )
            w1_row[0] += Kj
            blocks.append(blk)
        st = dict(H=H, W=W, Ms=H * W, Msp=max(128, H * W), C_in=C_in,
                  C_out=C_out, G0=G0, CP=CP, Cop=Cop, blocks=blocks,
                  r_sT=add(CP), r_tT=add(CP), r_wT=add(C_out), r_rw=add(C_out),
                  v_sT=vadd(CP), v_tT=vadd(CP))
        st["NBM"] = NB * st["Msp"]
        stages.append(st)
    fin = dict(r_fw=add(_HIDDEN), r_fb=add(_HIDDEN), r_pe=add(_HIDDEN))
    return stages, fin, w1_row[0], vec_row[0]


_STAGES, _FIN, _W1_ROWS, _VEC_ROWS = _layouts()
_MO_F = (_SPECS[-1][0] // 2) * (_SPECS[-1][1] // 2)

_TAPS = tuple((dy, dx) for dy in (-1, 0, 1) for dx in (-1, 0, 1))


def _np_masks(st):
    Msp, NBM, H, W = st["Msp"], st["NBM"], st["H"], st["W"]
    m = np.arange(NBM) % Msp
    yy, xx = m // W, m % W
    out = np.zeros((16, NBM), np.float32)
    for t, (dy, dx) in enumerate(_TAPS):
        out[t] = ((yy + dy >= 0) & (yy + dy < H) &
                  (xx + dx >= 0) & (xx + dx < W)).astype(np.float32)
    out[9] = 4.0 * ((yy % 2 == 0) & (xx % 2 == 0) & (yy < H))
    return out


def _np_pool(st, nrep):
    Msp, H, W = st["Msp"], st["H"], st["W"]
    Ho, Wo = H // 2, W // 2
    Mop = max(_MO_F, _rup(Ho * Wo, 128)) if st is not _STAGES[-1] else _MO_F
    p = np.zeros((Msp, Mop), np.float32)
    for m in range(H * W):
        py, px = m // W, m % W
        p[m, (py // 2) * Wo + (px // 2)] = 0.25
    return np.kron(np.eye(nrep, dtype=np.float32), p).astype(jnp.bfloat16)


_MASKS = [_np_masks(st) for st in _STAGES]
_POOL1 = _np_pool(_STAGES[0], 1)
_POOL2 = _np_pool(_STAGES[1], NB)
_POOL3 = _np_pool(_STAGES[2], NB)


def _dott(a, b):
    return lax.dot_general(a, b, (((0,), (0,)), ((), ())),
                           preferred_element_type=jnp.float32)


def _body(x_ref, w1t_ref, w2t_ref, ts1_ref, ts2_ref, ts3_ref, fwt_ref,
          vec_ref, peb_ref, p1_ref, p2_ref, p3_ref,
          m1_ref, m2_ref, m3_ref, o_ref, h_ref):
    ts_refs = (ts1_ref, ts2_ref, ts3_ref)
    mask_refs = (m1_ref, m2_ref, m3_ref)

    cur = x_ref[0]

    for si, st in enumerate(_STAGES):
        NBM, W, G0, CP, Cop = st["NBM"], st["W"], st["G0"], st["CP"], st["Cop"]
        mref = mask_refs[si]

        h_ref[pl.ds(0, G0), pl.ds(0, NBM)] = cur

        for j, blk in enumerate(st["blocks"]):
            Kj = blk["Kj"]
            h = h_ref[pl.ds(0, Kj), pl.ds(0, NBM)]
            s1 = vec_ref[pl.ds(blk["v_s1"], Kj), :]
            t1 = vec_ref[pl.ds(blk["v_t1"], Kj), :]
            a = jnp.maximum(h * s1 + t1, 0.0).astype(jnp.bfloat16)
            w1 = w1t_ref[pl.ds(blk["w1t"], Kj), :]
            u = _dott(w1, a)
            t2 = vec_ref[pl.ds(blk["v_t2"], _INTER), :]
            v = jnp.maximum(u + t2, 0.0).astype(jnp.bfloat16)
            w2 = w2t_ref[pl.ds(blk["w2t"], _INTER), :]
            p = _dott(w2, v)

            o = p[4 * _GROWTH:5 * _GROWTH, :]
            for t, (dy, dx) in enumerate(_TAPS):
                if t == 4:
                    continue
                s = dy * W + dx
                rows = p[t * _GROWTH:(t + 1) * _GROWTH, :]
                rows = pltpu.roll(rows, shift=(-s) % NBM, axis=1)
                o = o + rows * mref[t:t + 1, :]
            h_ref[pl.ds(G0 + j * _GROWTH, _GROWTH), pl.ds(0, NBM)] = (
                o.astype(jnp.bfloat16))

        h = h_ref[pl.ds(0, CP), pl.ds(0, NBM)]
        sT = vec_ref[pl.ds(st["v_sT"], CP), :]
        tT = vec_ref[pl.ds(st["v_tT"], CP), :]
        z = jnp.maximum(h * sT + tT, 0.0).astype(jnp.bfloat16)
        ts = ts_refs[si]
        y = _dott(ts[pl.ds(0, CP), :], z)
        idn = _dott(ts[pl.ds(CP, G0), :], cur)
        s_act = (y + idn * mref[9:10, :]).astype(jnp.bfloat16)

        if si == 0:
            parts = [jnp.dot(s_act[:, 1024 * i:1024 * (i + 1)], p1_ref[...],
                             preferred_element_type=jnp.float32)
                     for i in range(NB)]
            pooled = jnp.concatenate(parts, axis=1)
        elif si == 1:
            pooled = jnp.dot(s_act, p2_ref[...],
                             preferred_element_type=jnp.float32)
        else:
            pooled = jnp.dot(s_act, p3_ref[...],
                             preferred_element_type=jnp.float32)
        cur = jnp.maximum(pooled, 0.0).astype(jnp.bfloat16)

    out = _dott(fwt_ref[...], cur)
    o_ref[0] = out + peb_ref[...]


def kernel(x_nchw, slab):
    f32, bf16 = jnp.float32, jnp.bfloat16
    G = _B // NB

    x = x_nchw.reshape(G, NB, 3, 1024)
    x = jnp.pad(x, ((0, 0), (0, 0), (0, _STAGES[0]["G0"] - 3), (0, 0)))
    xb = jnp.transpose(x, (0, 2, 1, 3)).reshape(G, _STAGES[0]["G0"],
                                                NB * 1024).astype(bf16)

    w1t_parts, w2t_parts, ts_ops, vec_parts = [], [], [], []
    for st in _STAGES:
        CP, Cop, G0, C_in, C_out = (st["CP"], st["Cop"], st["G0"],
                                    st["C_in"], st["C_out"])
        for blk in st["blocks"]:
            Kj = blk["Kj"]
            w1 = slab[blk["r_w1"]:blk["r_w1"] + _INTER, :Kj]
            s2 = slab[blk["r_s2"]:blk["r_s2"] + _INTER, :1]
            w1t_parts.append(jnp.transpose(w1 * s2))
            w2 = slab[blk["r_w2"]:blk["r_w2"] + 9 * _GROWTH, :_INTER]
            w2t_parts.append(jnp.transpose(w2))
            vec_parts += [slab[blk["r_s1"]:blk["r_s1"] + Kj, :1],
                          slab[blk["r_t1"]:blk["r_t1"] + Kj, :1],
                          slab[blk["r_t2"]:blk["r_t2"] + _INTER, :1]]
        wT = slab[st["r_wT"]:st["r_wT"] + C_out, :CP]
        rw = slab[st["r_rw"]:st["r_rw"] + C_out, :C_in]
        wTt = jnp.pad(jnp.transpose(wT), ((0, 0), (0, Cop - C_out)))
        rwt = jnp.pad(jnp.transpose(rw),
                      ((0, G0 - C_in), (0, Cop - C_out)))
        ts_ops.append(jnp.concatenate([wTt, rwt], axis=0).astype(bf16))
        vec_parts += [slab[st["r_sT"]:st["r_sT"] + CP, :1],
                      slab[st["r_tT"]:st["r_tT"] + CP, :1]]

    w1t = jnp.concatenate(w1t_parts, axis=0).astype(bf16)
    w2t = jnp.concatenate(w2t_parts, axis=0).astype(bf16)
    vec = jnp.concatenate(vec_parts, axis=0).astype(f32)
    fw = slab[_FIN["r_fw"]:_FIN["r_fw"] + _HIDDEN, :_SPECS[-1][3]]
    fwt = jnp.transpose(fw).astype(bf16)
    fb = slab[_FIN["r_fb"]:_FIN["r_fb"] + _HIDDEN, :1]
    pe = slab[_FIN["r_pe"]:_FIN["r_pe"] + _HIDDEN, :_MO_F]
    peb = (jnp.tile(pe, (1, NB)) + fb).astype(f32)

    const = lambda shape: pl.BlockSpec(shape, lambda b: (0,) * len(shape))
    out = pl.pallas_call(
        _body,
        out_shape=jax.ShapeDtypeStruct((G, _HIDDEN, NB * _MO_F), f32),
        grid_spec=pltpu.PrefetchScalarGridSpec(
            num_scalar_prefetch=0,
            grid=(G,),
            in_specs=[pl.BlockSpec((1, _STAGES[0]["G0"], NB * 1024),
                                   lambda b: (b, 0, 0)),
                      const(w1t.shape), const(w2t.shape),
                      const(ts_ops[0].shape), const(ts_ops[1].shape),
                      const(ts_ops[2].shape), const(fwt.shape),
                      const(vec.shape), const(peb.shape),
                      const(_POOL1.shape), const(_POOL2.shape),
                      const(_POOL3.shape),
                      const(_MASKS[0].shape), const(_MASKS[1].shape),
                      const(_MASKS[2].shape)],
            out_specs=pl.BlockSpec((1, _HIDDEN, NB * _MO_F),
                                   lambda b: (b, 0, 0)),
            scratch_shapes=[pltpu.VMEM((_STAGES[-1]["CP"],
                                        NB * 1024), bf16)]),
        compiler_params=pltpu.CompilerParams(
            dimension_semantics=("parallel",)),
    )(xb, w1t, w2t, ts_ops[0], ts_ops[1], ts_ops[2], fwt, vec, peb,
      _POOL1, _POOL2, _POOL3, _MASKS[0], _MASKS[1], _MASKS[2])

    out = out.reshape(G, _HIDDEN, NB, _MO_F)
    return jnp.transpose(out, (0, 2, 3, 1)).reshape(_B, _MO_F, _HIDDEN)


# speedup vs baseline: 3.4529x; 1.4897x over previous
import numpy as np
import jax
import jax.numpy as jnp
from jax import lax
from jax.experimental import pallas as pl
from jax.experimental.pallas import tpu as pltpu

NB = 8
_B = 1024
_GROWTH = 32
_INTER = 128

_SPECS = ((32, 32, 3, 65), (16, 16, 65, 96), (8, 8, 96, 112))
_HIDDEN = 256


def _rup(n, k):
    return -(-n // k) * k


def _layouts():
    row = [0]

    def add(r):
        off = row[0]
        row[0] += _rup(r, 8)
        return off

    w1_row = [0]
    vec_row = [0]

    def vadd(r):
        off = vec_row[0]
        vec_row[0] += r
        return off

    stages = []
    for si, (H, W, C_in, C_out) in enumerate(_SPECS):
        G0 = _rup(C_in, 8)
        CP = G0 + 4 * _GROWTH
        Cop = _rup(C_out, 8)
        blocks = []
        for j in range(4):
            Kj = G0 + j * _GROWTH
            blk = dict(
                Kj=Kj,
                r_s1=add(CP), r_t1=add(CP), r_w1=add(_INTER),
                r_s2=add(_INTER), r_t2=add(_INTER), r_w2=add(9 * _GROWTH),
                w1t=w1_row[0], w2t=_INTER * (si * 4 + j),
                v_s1=vadd(Kj), v_t1=vadd(Kj), v_t2=vadd(_INTER),
            )
            w1_row[0] += Kj
            blocks.append(blk)
        st = dict(H=H, W=W, Ms=H * W, Msp=max(128, H * W), C_in=C_in,
                  C_out=C_out, G0=G0, CP=CP, Cop=Cop, blocks=blocks,
                  r_sT=add(CP), r_tT=add(CP), r_wT=add(C_out), r_rw=add(C_out),
                  v_sT=vadd(CP), v_tT=vadd(CP))
        st["NBM"] = NB * st["Msp"]
        stages.append(st)
    fin = dict(r_fw=add(_HIDDEN), r_fb=add(_HIDDEN), r_pe=add(_HIDDEN))
    return stages, fin, w1_row[0], vec_row[0]


_STAGES, _FIN, _W1_ROWS, _VEC_ROWS = _layouts()
_MO_F = (_SPECS[-1][0] // 2) * (_SPECS[-1][1] // 2)

_TAPS = tuple((dy, dx) for dy in (-1, 0, 1) for dx in (-1, 0, 1))


def _np_masks(st):
    Msp, NBM, H, W = st["Msp"], st["NBM"], st["H"], st["W"]
    m = np.arange(NBM) % Msp
    yy, xx = m // W, m % W
    out = np.zeros((16, NBM), np.float32)
    for t, (dy, dx) in enumerate(_TAPS):
        out[t] = ((yy + dy >= 0) & (yy + dy < H) &
                  (xx + dx >= 0) & (xx + dx < W)).astype(np.float32)
    out[9] = 4.0 * ((yy % 2 == 0) & (xx % 2 == 0) & (yy < H))
    return out.astype(jnp.bfloat16)


def _np_pool(st, nrep):
    Msp, H, W = st["Msp"], st["H"], st["W"]
    Ho, Wo = H // 2, W // 2
    Mop = max(_MO_F, _rup(Ho * Wo, 128)) if st is not _STAGES[-1] else _MO_F
    p = np.zeros((Msp, Mop), np.float32)
    for m in range(H * W):
        py, px = m // W, m % W
        p[m, (py // 2) * Wo + (px // 2)] = 0.25
    return np.kron(np.eye(nrep, dtype=np.float32), p).astype(jnp.bfloat16)


_MASKS = [_np_masks(st) for st in _STAGES]
_POOL1 = _np_pool(_STAGES[0], 1)
_POOL2 = _np_pool(_STAGES[1], NB)
_POOL3 = _np_pool(_STAGES[2], NB)


def _dott(a, b):
    return lax.dot_general(a, b, (((0,), (0,)), ((), ())),
                           preferred_element_type=jnp.float32)


def _body(x_ref, w1t_ref, w2t_ref, ts1_ref, ts2_ref, ts3_ref, fwt_ref,
          vec_ref, peb_ref, p1_ref, p2_ref, p3_ref,
          m1_ref, m2_ref, m3_ref, o_ref, h_ref):
    ts_refs = (ts1_ref, ts2_ref, ts3_ref)
    mask_refs = (m1_ref, m2_ref, m3_ref)

    cur = x_ref[0]

    for si, st in enumerate(_STAGES):
        NBM, W, G0, CP, Cop = st["NBM"], st["W"], st["G0"], st["CP"], st["Cop"]
        mref = mask_refs[si]

        h_ref[pl.ds(0, G0), pl.ds(0, NBM)] = cur

        for j, blk in enumerate(st["blocks"]):
            Kj = blk["Kj"]
            h = h_ref[pl.ds(0, Kj), pl.ds(0, NBM)]
            s1 = vec_ref[pl.ds(blk["v_s1"], Kj), :]
            t1 = vec_ref[pl.ds(blk["v_t1"], Kj), :]
            a = jnp.maximum(h * s1 + t1, 0)
            w1 = w1t_ref[pl.ds(blk["w1t"], Kj), :]
            u = _dott(w1, a)
            t2 = vec_ref[pl.ds(blk["v_t2"], _INTER), :]
            v = jnp.maximum(u.astype(jnp.bfloat16) + t2, 0)
            w2 = w2t_ref[pl.ds(blk["w2t"], _INTER), :]
            p = _dott(w2, v).astype(jnp.bfloat16)

            o = p[4 * _GROWTH:5 * _GROWTH, :]
            for t, (dy, dx) in enumerate(_TAPS):
                if t == 4:
                    continue
                s = dy * W + dx
                rows = p[t * _GROWTH:(t + 1) * _GROWTH, :]
                rows = pltpu.roll(rows, shift=(-s) % NBM, axis=1)
                o = o + rows * mref[t:t + 1, :]
            h_ref[pl.ds(G0 + j * _GROWTH, _GROWTH), pl.ds(0, NBM)] = o

        h = h_ref[pl.ds(0, CP), pl.ds(0, NBM)]
        sT = vec_ref[pl.ds(st["v_sT"], CP), :]
        tT = vec_ref[pl.ds(st["v_tT"], CP), :]
        z = jnp.maximum(h * sT + tT, 0)
        ts = ts_refs[si]
        y = _dott(ts[pl.ds(0, CP), :], z)
        idn = _dott(ts[pl.ds(CP, G0), :], cur)
        s_act = (y + idn * mref[9:10, :]).astype(jnp.bfloat16)

        if si == 0:
            parts = [jnp.dot(s_act[:, 1024 * i:1024 * (i + 1)], p1_ref[...],
                             preferred_element_type=jnp.float32)
                     for i in range(NB)]
            pooled = jnp.concatenate(parts, axis=1)
        elif si == 1:
            pooled = jnp.dot(s_act, p2_ref[...],
                             preferred_element_type=jnp.float32)
        else:
            pooled = jnp.dot(s_act, p3_ref[...],
                             preferred_element_type=jnp.float32)
        cur = jnp.maximum(pooled, 0.0).astype(jnp.bfloat16)

    out = _dott(fwt_ref[...], cur)
    o_ref[0] = out + peb_ref[...]


def kernel(x_nchw, slab):
    f32, bf16 = jnp.float32, jnp.bfloat16
    G = _B // NB

    x = x_nchw.reshape(G, NB, 3, 1024)
    x = jnp.pad(x, ((0, 0), (0, 0), (0, _STAGES[0]["G0"] - 3), (0, 0)))
    xb = jnp.transpose(x, (0, 2, 1, 3)).reshape(G, _STAGES[0]["G0"],
                                                NB * 1024).astype(bf16)

    w1t_parts, w2t_parts, ts_ops, vec_parts = [], [], [], []
    for st in _STAGES:
        CP, Cop, G0, C_in, C_out = (st["CP"], st["Cop"], st["G0"],
                                    st["C_in"], st["C_out"])
        for blk in st["blocks"]:
            Kj = blk["Kj"]
            w1 = slab[blk["r_w1"]:blk["r_w1"] + _INTER, :Kj]
            s2 = slab[blk["r_s2"]:blk["r_s2"] + _INTER, :1]
            w1t_parts.append(jnp.transpose(w1 * s2))
            w2 = slab[blk["r_w2"]:blk["r_w2"] + 9 * _GROWTH, :_INTER]
            w2t_parts.append(jnp.transpose(w2))
            vec_parts += [slab[blk["r_s1"]:blk["r_s1"] + Kj, :1],
                          slab[blk["r_t1"]:blk["r_t1"] + Kj, :1],
                          slab[blk["r_t2"]:blk["r_t2"] + _INTER, :1]]
        wT = slab[st["r_wT"]:st["r_wT"] + C_out, :CP]
        rw = slab[st["r_rw"]:st["r_rw"] + C_out, :C_in]
        wTt = jnp.pad(jnp.transpose(wT), ((0, 0), (0, Cop - C_out)))
        rwt = jnp.pad(jnp.transpose(rw),
                      ((0, G0 - C_in), (0, Cop - C_out)))
        ts_ops.append(jnp.concatenate([wTt, rwt], axis=0).astype(bf16))
        vec_parts += [slab[st["r_sT"]:st["r_sT"] + CP, :1],
                      slab[st["r_tT"]:st["r_tT"] + CP, :1]]

    w1t = jnp.concatenate(w1t_parts, axis=0).astype(bf16)
    w2t = jnp.concatenate(w2t_parts, axis=0).astype(bf16)
    vec = jnp.concatenate(vec_parts, axis=0).astype(bf16)
    fw = slab[_FIN["r_fw"]:_FIN["r_fw"] + _HIDDEN, :_SPECS[-1][3]]
    fwt = jnp.transpose(fw).astype(bf16)
    fb = slab[_FIN["r_fb"]:_FIN["r_fb"] + _HIDDEN, :1]
    pe = slab[_FIN["r_pe"]:_FIN["r_pe"] + _HIDDEN, :_MO_F]
    peb = (jnp.tile(pe, (1, NB)) + fb).astype(f32)

    const = lambda shape: pl.BlockSpec(shape, lambda b: (0,) * len(shape))
    out = pl.pallas_call(
        _body,
        out_shape=jax.ShapeDtypeStruct((G, _HIDDEN, NB * _MO_F), f32),
        grid_spec=pltpu.PrefetchScalarGridSpec(
            num_scalar_prefetch=0,
            grid=(G,),
            in_specs=[pl.BlockSpec((1, _STAGES[0]["G0"], NB * 1024),
                                   lambda b: (b, 0, 0)),
                      const(w1t.shape), const(w2t.shape),
                      const(ts_ops[0].shape), const(ts_ops[1].shape),
                      const(ts_ops[2].shape), const(fwt.shape),
                      const(vec.shape), const(peb.shape),
                      const(_POOL1.shape), const(_POOL2.shape),
                      const(_POOL3.shape),
                      const(_MASKS[0].shape), const(_MASKS[1].shape),
                      const(_MASKS[2].shape)],
            out_specs=pl.BlockSpec((1, _HIDDEN, NB * _MO_F),
                                   lambda b: (b, 0, 0)),
            scratch_shapes=[pltpu.VMEM((_STAGES[-1]["CP"],
                                        NB * 1024), bf16)]),
        compiler_params=pltpu.CompilerParams(
            dimension_semantics=("parallel",)),
    )(xb, w1t, w2t, ts_ops[0], ts_ops[1], ts_ops[2], fwt, vec, peb,
      _POOL1, _POOL2, _POOL3, _MASKS[0], _MASKS[1], _MASKS[2])

    out = out.reshape(G, _HIDDEN, NB, _MO_F)
    return jnp.transpose(out, (0, 2, 3, 1)).reshape(_B, _MO_F, _HIDDEN)
